# baseline (speedup 1.0000x reference)
import jax
import jax.numpy as jnp
from jax import lax
from jax.experimental import pallas as pl
from jax.experimental.pallas import tpu as pltpu
from jax.experimental.pallas import tpu_sc as plsc

NC = 2
NS = 16
L = 16
NW = NC * NS

BATCH = 16384
D = 32
BPW = BATCH // NW
NCHUNK = 4
CHUNK = BPW // NCHUNK
NGROUP = BPW // L


def _gmf_body(x_hbm, user_hbm, item_hbm, fcw_hbm, out_hbm,
              xv, u_idx, i_idx, u_rows, i_rows, wv, trans, res, sem):
    wid = lax.axis_index("s") * NC + lax.axis_index("c")
    base = wid * BPW

    pltpu.sync_copy(x_hbm.at[pl.ds(base * 2, BPW * 2)], xv)
    pltpu.sync_copy(fcw_hbm.at[0], wv)

    lane = lax.iota(jnp.int32, L)

    for g in range(NGROUP):
        flat = lane * 2 + (g * L * 2)
        u = plsc.load_gather(xv, [flat])
        i = plsc.load_gather(xv, [flat + 1])
        c, off = (g * L) // CHUNK, (g * L) % CHUNK
        u_idx[c, pl.ds(off, L)] = u
        i_idx[c, pl.ds(off, L)] = i

    copies = []
    for c in range(NCHUNK):
        copies.append(pltpu.async_copy(
            user_hbm.at[u_idx.at[c]], u_rows.at[pl.ds(c * CHUNK, CHUNK)], sem))
        copies.append(pltpu.async_copy(
            item_hbm.at[i_idx.at[c]], i_rows.at[pl.ds(c * CHUNK, CHUNK)], sem))
    for cp in copies:
        cp.wait()

    w0 = wv[pl.ds(0, L)]
    w1 = wv[pl.ds(L, L)]
    lane17 = lane * 17

    def group(g, carry):
        for r in range(L):
            row = g * L + r
            u0 = u_rows[row, pl.ds(0, L)]
            u1 = u_rows[row, pl.ds(L, L)]
            i0 = i_rows[row, pl.ds(0, L)]
            i1 = i_rows[row, pl.ds(L, L)]
            s = u0 * i0 * w0 + u1 * i1 * w1
            trans[pl.ds(r * 17, L)] = s
        acc = plsc.load_gather(trans, [lane17])
        for d in range(1, L):
            acc = acc + plsc.load_gather(trans, [lane17 + d])
        sig = 1.0 / (1.0 + jnp.exp(-acc))
        res[pl.ds(g * L, L)] = sig
        return carry

    lax.fori_loop(0, NGROUP, group, 0)

    pltpu.sync_copy(res, out_hbm.at[pl.ds(base, BPW)])


@jax.jit
def _gmf(x, user_table, item_table, fc_w):
    mesh = plsc.VectorSubcoreMesh(
        core_axis_name="c", subcore_axis_name="s", num_cores=NC, num_subcores=NS)
    fn = pl.kernel(
        _gmf_body,
        out_type=jax.ShapeDtypeStruct((BATCH,), jnp.float32),
        mesh=mesh,
        compiler_params=pltpu.CompilerParams(
            needs_layout_passes=False, use_tc_tiling_on_sc=False),
        scratch_types=[
            pltpu.VMEM((BPW * 2,), jnp.int32),
            pltpu.VMEM((NCHUNK, CHUNK), jnp.int32),
            pltpu.VMEM((NCHUNK, CHUNK), jnp.int32),
            pltpu.VMEM((BPW, D), jnp.float32),
            pltpu.VMEM((BPW, D), jnp.float32),
            pltpu.VMEM((D,), jnp.float32),
            pltpu.VMEM((L * 17,), jnp.float32),
            pltpu.VMEM((BPW,), jnp.float32),
            pltpu.SemaphoreType.DMA,
        ],
    )
    return fn(x.reshape(BATCH * 2), user_table, item_table, fc_w)


def kernel(x, user_table, item_table, fc_w):
    out = _gmf(x, user_table, item_table, fc_w)
    return out.reshape(BATCH, 1)


# speedup vs baseline: 3.4565x; 3.4565x over previous
import jax
import jax.numpy as jnp
from jax import lax
from jax.experimental import pallas as pl
from jax.experimental.pallas import tpu as pltpu
from jax.experimental.pallas import tpu_sc as plsc

NC = 2
NS = 16
L = 16
NW = NC * NS

BATCH = 16384
D = 32

BPW = BATCH // NW
CID = 4
NCHUNK = BPW // CID
WROW = 128


def _gmf_body(x_hbm, user_hbm, item_hbm, fcw_hbm, out_hbm,
              xs, xsh, uwin, iwin, wv, trans, res, sem0, sem1):
    wid = lax.axis_index("s") * NC + lax.axis_index("c")
    base = wid * BPW

    sid = lax.axis_index("s")
    pltpu.sync_copy(x_hbm.at[pl.ds(base * 2, BPW * 2)], xsh.at[sid])
    pltpu.sync_copy(xsh.at[sid], xs.at[pl.ds(0, BPW * 2)])
    pltpu.sync_copy(fcw_hbm.at[0], wv)

    for j in range(2 * CID):
        xs[BPW * 2 + j] = 0

    lane = lax.iota(jnp.int32, L)
    lane17 = lane * 17
    w0 = wv[pl.ds(0, L)]
    w1 = wv[pl.ds(L, L)]

    def fire(c, buf, sem):
        for k in range(CID):
            uid = xs[(c * CID + k) * 2]
            iid = xs[(c * CID + k) * 2 + 1]
            uoff = pl.multiple_of((uid >> 7) << 7, 128)
            ioff = pl.multiple_of((iid >> 7) << 7, 128)
            row = (buf * CID + k) * D
            pltpu.async_copy(
                user_hbm.at[:, pl.ds(uoff, 128)],
                uwin.at[pl.ds(row, D), pl.ds(0, 128)], sem)
            pltpu.async_copy(
                item_hbm.at[:, pl.ds(ioff, 128)],
                iwin.at[pl.ds(row, D), pl.ds(0, 128)], sem)

    def drain(sem):
        for k in range(CID):
            pltpu.make_async_copy(
                user_hbm.at[:, pl.ds(0, 128)],
                uwin.at[pl.ds(k * D, D), pl.ds(0, 128)], sem).wait()
            pltpu.make_async_copy(
                item_hbm.at[:, pl.ds(0, 128)],
                iwin.at[pl.ds(k * D, D), pl.ds(0, 128)], sem).wait()

    def compute(c, buf):
        for k in range(CID):
            ul = xs[(c * CID + k) * 2] & 127
            il = xs[(c * CID + k) * 2 + 1] & 127
            rows0 = (buf * CID + k) * D + lane
            rows1 = rows0 + L
            u0 = plsc.load_gather(uwin, [rows0, lane * 0 + ul])
            u1 = plsc.load_gather(uwin, [rows1, lane * 0 + ul])
            i0 = plsc.load_gather(iwin, [rows0, lane * 0 + il])
            i1 = plsc.load_gather(iwin, [rows1, lane * 0 + il])
            s = u0 * i0 * w0 + u1 * i1 * w1
            trans[pl.ds(((c & 3) * CID + k) * 17, L)] = s

        @pl.when((c & 3) == 3)
        def _():
            acc = plsc.load_gather(trans, [lane17])
            for d in range(1, L):
                acc = acc + plsc.load_gather(trans, [lane17 + d])
            res[pl.ds((c >> 2) * L, L)] = 1.0 / (1.0 + jnp.exp(-acc))

    fire(0, 0, sem0)

    def pair(c2, carry):
        c = 2 * c2
        fire(c + 1, 1, sem1)
        drain(sem0)
        compute(c, 0)
        fire(c + 2, 0, sem0)
        drain(sem1)
        compute(c + 1, 1)
        return carry

    lax.fori_loop(0, NCHUNK // 2, pair, 0)
    drain(sem0)

    pltpu.sync_copy(res, out_hbm.at[pl.ds(base, BPW)])


@jax.jit
def _gmf(x, user_t, item_t, fc_w):
    mesh = plsc.VectorSubcoreMesh(
        core_axis_name="c", subcore_axis_name="s", num_cores=NC, num_subcores=NS)
    fn = pl.kernel(
        _gmf_body,
        out_type=jax.ShapeDtypeStruct((BATCH,), jnp.float32),
        mesh=mesh,
        compiler_params=pltpu.CompilerParams(
            needs_layout_passes=False, use_tc_tiling_on_sc=True),
        scratch_types=[
            pltpu.SMEM((BPW * 2 + 2 * CID,), jnp.int32),
            pltpu.VMEM_SHARED((NS, BPW * 2), jnp.int32),
            pltpu.VMEM((2 * CID * D, WROW), jnp.float32),
            pltpu.VMEM((2 * CID * D, WROW), jnp.float32),
            pltpu.VMEM((D,), jnp.float32),
            pltpu.VMEM((L * 17,), jnp.float32),
            pltpu.VMEM((BPW,), jnp.float32),
            pltpu.SemaphoreType.DMA,
            pltpu.SemaphoreType.DMA,
        ],
    )
    return fn(x.reshape(BATCH * 2), user_t, item_t, fc_w)


def kernel(x, user_table, item_table, fc_w):
    out = _gmf(x, user_table.T, item_table.T, fc_w)
    return out.reshape(BATCH, 1)


# speedup vs baseline: 3.7503x; 1.0850x over previous
import jax
import jax.numpy as jnp
from jax import lax
from jax.experimental import pallas as pl
from jax.experimental.pallas import tpu as pltpu
from jax.experimental.pallas import tpu_sc as plsc

NC = 2
NS = 16
L = 16
NW = NC * NS

BATCH = 16384
D = 32

BPW = BATCH // NW
CID = 4
NCHUNK = BPW // CID
WROW = 128


def _gmf_body(x_hbm, user_hbm, item_hbm, fcw_hbm, out_hbm,
              xs, xsh, uwin, iwin, wv, trans, res, sem0, sem1, sem2):
    wid = lax.axis_index("s") * NC + lax.axis_index("c")
    base = wid * BPW

    sid = lax.axis_index("s")
    pltpu.sync_copy(x_hbm.at[pl.ds(base * 2, BPW * 2)], xsh.at[sid])
    pltpu.sync_copy(xsh.at[sid], xs.at[pl.ds(0, BPW * 2)])
    pltpu.sync_copy(fcw_hbm.at[0], wv)

    lane = lax.iota(jnp.int32, L)
    lane17 = lane * 17
    w0 = wv[pl.ds(0, L)]
    w1 = wv[pl.ds(L, L)]

    def fire(c, buf, sem):
        for k in range(CID):
            uid = xs[(c * CID + k) * 2]
            iid = xs[(c * CID + k) * 2 + 1]
            uoff = pl.multiple_of((uid >> 7) << 7, 128)
            ioff = pl.multiple_of((iid >> 7) << 7, 128)
            row = (buf * CID + k) * D
            pltpu.async_copy(
                user_hbm.at[:, pl.ds(uoff, 128)],
                uwin.at[pl.ds(row, D), pl.ds(0, 128)], sem)
            pltpu.async_copy(
                item_hbm.at[:, pl.ds(ioff, 128)],
                iwin.at[pl.ds(row, D), pl.ds(0, 128)], sem)

    def drain(sem):
        for k in range(CID):
            pltpu.make_async_copy(
                user_hbm.at[:, pl.ds(0, 128)],
                uwin.at[pl.ds(k * D, D), pl.ds(0, 128)], sem).wait()
            pltpu.make_async_copy(
                item_hbm.at[:, pl.ds(0, 128)],
                iwin.at[pl.ds(k * D, D), pl.ds(0, 128)], sem).wait()

    def compute(c, buf):
        for k in range(CID):
            ul = xs[(c * CID + k) * 2] & 127
            il = xs[(c * CID + k) * 2 + 1] & 127
            rows0 = (buf * CID + k) * D + lane
            rows1 = rows0 + L
            u0 = plsc.load_gather(uwin, [rows0, lane * 0 + ul])
            u1 = plsc.load_gather(uwin, [rows1, lane * 0 + ul])
            i0 = plsc.load_gather(iwin, [rows0, lane * 0 + il])
            i1 = plsc.load_gather(iwin, [rows1, lane * 0 + il])
            s = u0 * i0 * w0 + u1 * i1 * w1
            trans[pl.ds(((c & 3) * CID + k) * 17, L)] = s

        @pl.when((c & 3) == 3)
        def _():
            acc = plsc.load_gather(trans, [lane17])
            for d in range(1, L):
                acc = acc + plsc.load_gather(trans, [lane17 + d])
            res[pl.ds((c >> 2) * L, L)] = 1.0 / (1.0 + jnp.exp(-acc))

    fire(0, 0, sem0)
    fire(1, 1, sem1)

    def triple(t, carry):
        c = 3 * t
        fire(c + 2, 2, sem2)
        drain(sem0)
        compute(c, 0)
        fire(c + 3, 0, sem0)
        drain(sem1)
        compute(c + 1, 1)
        fire(c + 4, 1, sem1)
        drain(sem2)
        compute(c + 2, 2)
        return carry

    lax.fori_loop(0, (NCHUNK - 2) // 3, triple, 0)
    drain(sem0)
    compute(NCHUNK - 2, 0)
    drain(sem1)
    compute(NCHUNK - 1, 1)

    pltpu.sync_copy(res, out_hbm.at[pl.ds(base, BPW)])


@jax.jit
def _gmf(x, user_t, item_t, fc_w):
    mesh = plsc.VectorSubcoreMesh(
        core_axis_name="c", subcore_axis_name="s", num_cores=NC, num_subcores=NS)
    fn = pl.kernel(
        _gmf_body,
        out_type=jax.ShapeDtypeStruct((BATCH,), jnp.float32),
        mesh=mesh,
        compiler_params=pltpu.CompilerParams(
            needs_layout_passes=False, use_tc_tiling_on_sc=True),
        scratch_types=[
            pltpu.SMEM((BPW * 2 + 2 * CID,), jnp.int32),
            pltpu.VMEM_SHARED((NS, BPW * 2), jnp.int32),
            pltpu.VMEM((3 * CID * D, WROW), jnp.float32),
            pltpu.VMEM((3 * CID * D, WROW), jnp.float32),
            pltpu.VMEM((D,), jnp.float32),
            pltpu.VMEM((L * 17,), jnp.float32),
            pltpu.VMEM((BPW,), jnp.float32),
            pltpu.SemaphoreType.DMA,
            pltpu.SemaphoreType.DMA,
            pltpu.SemaphoreType.DMA,
        ],
    )
    return fn(x.reshape(BATCH * 2), user_t, item_t, fc_w)


def kernel(x, user_table, item_table, fc_w):
    out = _gmf(x, user_table.T, item_table.T, fc_w)
    return out.reshape(BATCH, 1)


# speedup vs baseline: 4.1467x; 1.1057x over previous
import jax
import jax.numpy as jnp
from jax import lax
from jax.experimental import pallas as pl
from jax.experimental.pallas import tpu as pltpu
from jax.experimental.pallas import tpu_sc as plsc

NC = 2
NS = 16
L = 16
NW = NC * NS

BATCH = 16384
D = 32

BPW = BATCH // NW
CID = 2
NCHUNK = BPW // CID
WROW = 128


def _gmf_body(x_hbm, user_hbm, item_hbm, fcw_hbm, out_hbm,
              xs, xsh, uwin, iwin, wv, trans, res, sem0, sem1, sem2, sem3, sem4, sem5):
    wid = lax.axis_index("s") * NC + lax.axis_index("c")
    base = wid * BPW

    sid = lax.axis_index("s")
    pltpu.sync_copy(x_hbm.at[pl.ds(base * 2, BPW * 2)], xsh.at[sid])
    pltpu.sync_copy(xsh.at[sid], xs.at[pl.ds(0, BPW * 2)])
    pltpu.sync_copy(fcw_hbm.at[0], wv)

    lane = lax.iota(jnp.int32, L)
    lane17 = lane * 17
    w0 = wv[pl.ds(0, L)]
    w1 = wv[pl.ds(L, L)]

    def fire(c, buf, sem):
        for k in range(CID):
            uid = xs[(c * CID + k) * 2]
            iid = xs[(c * CID + k) * 2 + 1]
            uoff = pl.multiple_of((uid >> 7) << 7, 128)
            ioff = pl.multiple_of((iid >> 7) << 7, 128)
            row = (buf * CID + k) * D
            pltpu.async_copy(
                user_hbm.at[:, pl.ds(uoff, 128)],
                uwin.at[pl.ds(row, D), pl.ds(0, 128)], sem)
            pltpu.async_copy(
                item_hbm.at[:, pl.ds(ioff, 128)],
                iwin.at[pl.ds(row, D), pl.ds(0, 128)], sem)

    def drain(sem):
        for k in range(CID):
            pltpu.make_async_copy(
                user_hbm.at[:, pl.ds(0, 128)],
                uwin.at[pl.ds(k * D, D), pl.ds(0, 128)], sem).wait()
            pltpu.make_async_copy(
                item_hbm.at[:, pl.ds(0, 128)],
                iwin.at[pl.ds(k * D, D), pl.ds(0, 128)], sem).wait()

    def compute(c, buf):
        for k in range(CID):
            ul = xs[(c * CID + k) * 2] & 127
            il = xs[(c * CID + k) * 2 + 1] & 127
            rows0 = (buf * CID + k) * D + lane
            rows1 = rows0 + L
            u0 = plsc.load_gather(uwin, [rows0, lane * 0 + ul])
            u1 = plsc.load_gather(uwin, [rows1, lane * 0 + ul])
            i0 = plsc.load_gather(iwin, [rows0, lane * 0 + il])
            i1 = plsc.load_gather(iwin, [rows1, lane * 0 + il])
            s = u0 * i0 * w0 + u1 * i1 * w1
            trans[pl.ds(((c & 7) * CID + k) * 17, L)] = s

        @pl.when((c & 7) == 7)
        def _():
            acc = plsc.load_gather(trans, [lane17])
            for d in range(1, L):
                acc = acc + plsc.load_gather(trans, [lane17 + d])
            res[pl.ds((c >> 3) * L, L)] = 1.0 / (1.0 + jnp.exp(-acc))

    sems = (sem0, sem1, sem2, sem3, sem4, sem5)
    for m in range(5):
        fire(m, m, sems[m])

    def sextet(t, carry):
        c = 6 * t
        for j in range(6):
            fire(c + 5 + j, (5 + j) % 6, sems[(5 + j) % 6])
            drain(sems[j])
            compute(c + j, j)
        return carry

    nloop = (NCHUNK - 10) // 6
    lax.fori_loop(0, nloop, sextet, 0)
    cbase = nloop * 6
    for m in range(cbase, NCHUNK):
        if m + 5 < NCHUNK:
            fire(m + 5, (m + 5) % 6, sems[(m + 5) % 6])
        drain(sems[m % 6])
        compute(m, m % 6)

    pltpu.sync_copy(res, out_hbm.at[pl.ds(base, BPW)])


@jax.jit
def _gmf(x, user_t, item_t, fc_w):
    mesh = plsc.VectorSubcoreMesh(
        core_axis_name="c", subcore_axis_name="s", num_cores=NC, num_subcores=NS)
    fn = pl.kernel(
        _gmf_body,
        out_type=jax.ShapeDtypeStruct((BATCH,), jnp.float32),
        mesh=mesh,
        compiler_params=pltpu.CompilerParams(
            needs_layout_passes=False, use_tc_tiling_on_sc=True),
        scratch_types=[
            pltpu.SMEM((BPW * 2 + 2 * CID,), jnp.int32),
            pltpu.VMEM_SHARED((NS, BPW * 2), jnp.int32),
            pltpu.VMEM((6 * CID * D, WROW), jnp.float32),
            pltpu.VMEM((6 * CID * D, WROW), jnp.float32),
            pltpu.VMEM((D,), jnp.float32),
            pltpu.VMEM((L * 17,), jnp.float32),
            pltpu.VMEM((BPW,), jnp.float32),
            pltpu.SemaphoreType.DMA,
            pltpu.SemaphoreType.DMA,
            pltpu.SemaphoreType.DMA,
            pltpu.SemaphoreType.DMA,
            pltpu.SemaphoreType.DMA,
            pltpu.SemaphoreType.DMA,
        ],
    )
    return fn(x.reshape(BATCH * 2), user_t, item_t, fc_w)


def kernel(x, user_table, item_table, fc_w):
    out = _gmf(x, user_table.T, item_table.T, fc_w)
    return out.reshape(BATCH, 1)
